# Initial kernel scaffold; baseline (speedup 1.0000x reference)
import jax
import jax.numpy as jnp
from jax.experimental import pallas as pl


def kernel(species, edge_src, edge_dst, distances, switch, W_ud_0, W_rb_0, W_mix_0, b_mix_0, W_ud_1, W_rb_1, W_mix_1, b_mix_1):
    raise NotImplementedError("write your pallas kernel here")


# speedup vs baseline: 2.3636x; 2.3636x over previous
import functools

import jax
import jax.numpy as jnp
from jax import lax
from jax.experimental import pallas as pl
from jax.experimental.pallas import tpu as pltpu
from jax.experimental.pallas import tpu_sc as plsc

N = 10000
E = 320000
DIM = 128
DIM_SRC = 64
DIM_DST = 32
UDW = DIM_SRC + DIM_DST
ZMAX = 87
RB = 8
CUTOFF = 5.0

NC, NS, LANES = 2, 16, 16
NW = NC * NS
CH = 128
CPT = 80
EPT = CH * CPT
EP = NW * EPT
NP = 10240
ZR = 128

_HIGH = jax.lax.Precision.HIGHEST


def _prep_body(dist_ref, sw_ref, wrb0_ref, wrb1_ref, phi0_ref, phi1_ref):
    x = dist_ref[...]
    sw = sw_ref[...]
    n = (jax.lax.broadcasted_iota(jnp.int32, (1, RB), 1) + 1).astype(jnp.float32)
    arg = n * (jnp.pi / CUTOFF) * x
    rb = jnp.sqrt(2.0 / CUTOFF) * jnp.sin(arg) / jnp.maximum(x, 1e-3)
    swb = jnp.broadcast_to(sw, (x.shape[0], DIM_DST))
    phi0 = jnp.dot(rb, wrb0_ref[...], precision=_HIGH) * sw
    phi1 = jnp.dot(rb, wrb1_ref[...], precision=_HIGH) * sw
    phi0_ref[...] = jnp.concatenate([phi0, swb], axis=1)
    phi1_ref[...] = jnp.concatenate([phi1, swb], axis=1)


def _prep(dist_p, sw_p, W_rb_0, W_rb_1):
    B = 4096
    grid = (EP // B,)
    return pl.pallas_call(
        _prep_body,
        grid=grid,
        in_specs=[
            pl.BlockSpec((B, 1), lambda i: (i, 0)),
            pl.BlockSpec((B, 1), lambda i: (i, 0)),
            pl.BlockSpec((RB, DIM_SRC), lambda i: (0, 0)),
            pl.BlockSpec((RB, DIM_SRC), lambda i: (0, 0)),
        ],
        out_specs=[
            pl.BlockSpec((B, UDW), lambda i: (i, 0)),
            pl.BlockSpec((B, UDW), lambda i: (i, 0)),
        ],
        out_shape=[
            jax.ShapeDtypeStruct((EP, UDW), jnp.float32),
            jax.ShapeDtypeStruct((EP, UDW), jnp.float32),
        ],
    )(dist_p, sw_p, W_rb_0, W_rb_1)


def _ud0_body(spec_ref, wud_ref, out_ref):
    oh = (spec_ref[...] == jax.lax.broadcasted_iota(jnp.int32, (N, ZMAX), 1))
    out_ref[...] = jnp.dot(oh.astype(jnp.float32), wud_ref[...], precision=_HIGH)


def _ud0(species2d, W_ud_0):
    return pl.pallas_call(
        _ud0_body,
        out_shape=jax.ShapeDtypeStruct((N, UDW), jnp.float32),
    )(species2d, W_ud_0)


def _edge_body(ud_hbm, src_hbm, dst_hbm, phi_hbm, out_hbm,
               acc, srcb, dstb, rows, phib, zbuf):
    c = lax.axis_index("c")
    s = lax.axis_index("s")
    w = c * NS + s

    @pl.loop(0, ZR)
    def _(i):
        for jj in range(UDW // LANES):
            zbuf[i, pl.ds(jj * LANES, LANES)] = jnp.zeros((LANES,), jnp.float32)

    for k in range(NP // NS // ZR):
        pltpu.sync_copy(zbuf, acc.at[pl.ds(s * (NP // NS) + k * ZR, ZR)])
    plsc.subcore_barrier()

    @pl.loop(0, CPT)
    def _(i):
        e0 = w * EPT + i * CH
        pltpu.sync_copy(src_hbm.at[pl.ds(e0, CH)], srcb)
        pltpu.sync_copy(dst_hbm.at[pl.ds(e0, CH)], dstb)
        pltpu.sync_copy(phi_hbm.at[pl.ds(e0, CH)], phib)
        pltpu.sync_copy(ud_hbm.at[srcb], rows)

        @pl.loop(0, CH)
        def _(e):
            for jj in range(UDW // LANES):
                sl = pl.ds(jj * LANES, LANES)
                rows[e, sl] = rows[e, sl] * phib[e, sl]

        pltpu.sync_copy(rows, acc.at[dstb], add=True)

    plsc.subcore_barrier()
    for k in range(NP // NS // ZR):
        r0 = s * (NP // NS) + k * ZR
        pltpu.sync_copy(acc.at[pl.ds(r0, ZR)], out_hbm.at[c, pl.ds(r0, ZR)])


def _edge_sc(UD, src_p, dst_p, PHI):
    mesh = plsc.VectorSubcoreMesh(core_axis_name="c", subcore_axis_name="s")
    kern = pl.kernel(
        _edge_body,
        out_type=jax.ShapeDtypeStruct((NC, NP, UDW), jnp.float32),
        mesh=mesh,
        compiler_params=pltpu.CompilerParams(use_tc_tiling_on_sc=False),
        scratch_types=[
            pltpu.VMEM_SHARED((NP, UDW), jnp.float32),
            pltpu.VMEM((CH,), jnp.int32),
            pltpu.VMEM((CH,), jnp.int32),
            pltpu.VMEM((CH, UDW), jnp.float32),
            pltpu.VMEM((CH, UDW), jnp.float32),
            pltpu.VMEM((ZR, UDW), jnp.float32),
        ],
    )
    return kern(UD, src_p, dst_p, PHI)


def _silu_tssr2(x):
    out = x * jax.lax.logistic(x)
    ax = jnp.abs(out)
    return jnp.where(ax <= 1.0,
                     out,
                     jnp.sign(out) * (2.0 * jnp.sqrt(jnp.maximum(ax, 1.0)) - 1.0))


BN = 2000


def _dense0_body(spec_ref, m_ref, w0a_ref, w0b_ref, w0c_ref, b0_ref, wud1_ref,
                 xi1_ref, ud1_ref):
    oh = (spec_ref[...] == jax.lax.broadcasted_iota(jnp.int32, (BN, ZMAX), 1))
    msum = m_ref[0] + m_ref[1]
    pre = (jnp.dot(oh.astype(jnp.float32), w0a_ref[...], precision=_HIGH)
           + jnp.dot(msum[:, :DIM_SRC], w0b_ref[...], precision=_HIGH)
           + jnp.dot(msum[:, DIM_SRC:], w0c_ref[...], precision=_HIGH)
           + b0_ref[...])
    xi1 = _silu_tssr2(pre)
    xi1_ref[...] = xi1
    ud1_ref[...] = jnp.dot(xi1, wud1_ref[...], precision=_HIGH)


def _dense0(species2d, M0, W0a, W0b, W0c, b0, W_ud_1):
    return pl.pallas_call(
        _dense0_body,
        grid=(N // BN,),
        in_specs=[
            pl.BlockSpec((BN, 1), lambda i: (i, 0)),
            pl.BlockSpec((NC, BN, UDW), lambda i: (0, i, 0)),
            pl.BlockSpec((ZMAX, DIM), lambda i: (0, 0)),
            pl.BlockSpec((DIM_SRC, DIM), lambda i: (0, 0)),
            pl.BlockSpec((DIM_DST, DIM), lambda i: (0, 0)),
            pl.BlockSpec((1, DIM), lambda i: (0, 0)),
            pl.BlockSpec((DIM, UDW), lambda i: (0, 0)),
        ],
        out_specs=[
            pl.BlockSpec((BN, DIM), lambda i: (i, 0)),
            pl.BlockSpec((BN, UDW), lambda i: (i, 0)),
        ],
        out_shape=[
            jax.ShapeDtypeStruct((N, DIM), jnp.float32),
            jax.ShapeDtypeStruct((N, UDW), jnp.float32),
        ],
    )(species2d, M0, W0a, W0b, W0c, b0, W_ud_1)


def _dense1_body(xi_ref, m_ref, w1a_ref, w1b_ref, w1c_ref, b1_ref, out_ref):
    xi = xi_ref[...]
    msum = m_ref[0] + m_ref[1]
    pre = (jnp.dot(xi, w1a_ref[...], precision=_HIGH)
           + jnp.dot(msum[:, :DIM_SRC], w1b_ref[...], precision=_HIGH)
           + jnp.dot(msum[:, DIM_SRC:], w1c_ref[...], precision=_HIGH)
           + b1_ref[...])
    out_ref[...] = xi + _silu_tssr2(pre)


def _dense1(xi1, M1, W1a, W1b, W1c, b1):
    return pl.pallas_call(
        _dense1_body,
        grid=(N // BN,),
        in_specs=[
            pl.BlockSpec((BN, DIM), lambda i: (i, 0)),
            pl.BlockSpec((NC, BN, UDW), lambda i: (0, i, 0)),
            pl.BlockSpec((DIM, DIM), lambda i: (0, 0)),
            pl.BlockSpec((DIM_SRC, DIM), lambda i: (0, 0)),
            pl.BlockSpec((DIM_DST, DIM), lambda i: (0, 0)),
            pl.BlockSpec((1, DIM), lambda i: (0, 0)),
        ],
        out_specs=pl.BlockSpec((BN, DIM), lambda i: (i, 0)),
        out_shape=jax.ShapeDtypeStruct((N, DIM), jnp.float32),
    )(xi1, M1, W1a, W1b, W1c, b1)


def kernel(species, edge_src, edge_dst, distances, switch,
           W_ud_0, W_rb_0, W_mix_0, b_mix_0,
           W_ud_1, W_rb_1, W_mix_1, b_mix_1):
    pad = EP - E
    src_p = jnp.pad(edge_src.astype(jnp.int32), (0, pad))
    dst_p = jnp.pad(edge_dst.astype(jnp.int32), (0, pad))
    dist_p = jnp.pad(distances, (0, pad), constant_values=1.0)[:, None]
    sw_p = jnp.pad(switch, (0, pad))[:, None]
    species2d = species.astype(jnp.int32)[:, None]

    PHI0, PHI1 = _prep(dist_p, sw_p, W_rb_0, W_rb_1)
    UD0 = _ud0(species2d, W_ud_0)

    M0 = _edge_sc(UD0, src_p, dst_p, PHI0)
    xi1, UD1 = _dense0(species2d, M0,
                       W_mix_0[:ZMAX], W_mix_0[ZMAX:ZMAX + DIM_SRC],
                       W_mix_0[ZMAX + DIM_SRC:], b_mix_0[None, :], W_ud_1)

    M1 = _edge_sc(UD1, src_p, dst_p, PHI1)
    out = _dense1(xi1, M1,
                  W_mix_1[:DIM], W_mix_1[DIM:DIM + DIM_SRC],
                  W_mix_1[DIM + DIM_SRC:], b_mix_1[None, :])
    return out


# speedup vs baseline: 2.6087x; 1.1037x over previous
import functools

import jax
import jax.numpy as jnp
from jax import lax
from jax.experimental import pallas as pl
from jax.experimental.pallas import tpu as pltpu
from jax.experimental.pallas import tpu_sc as plsc

N = 10000
E = 320000
DIM = 128
DIM_SRC = 64
DIM_DST = 32
UDW = DIM_SRC + DIM_DST
ZMAX = 87
RB = 8
CUTOFF = 5.0

NC, NS, LANES = 2, 16, 16
NW = NC * NS
CH = 112
CPT = 92
EPT = CH * CPT
EP = NW * EPT
NP = 10240
ZR = 80

_HIGH = jax.lax.Precision.HIGHEST


def _prep_body(dist_ref, sw_ref, wrb0_ref, wrb1_ref, phi0_ref, phi1_ref):
    x = dist_ref[...]
    sw = sw_ref[...]
    n = (jax.lax.broadcasted_iota(jnp.int32, (1, RB), 1) + 1).astype(jnp.float32)
    arg = n * (jnp.pi / CUTOFF) * x
    rb = jnp.sqrt(2.0 / CUTOFF) * jnp.sin(arg) / jnp.maximum(x, 1e-3)
    swb = jnp.broadcast_to(sw, (x.shape[0], DIM_DST))
    phi0 = jnp.dot(rb, wrb0_ref[...], precision=_HIGH) * sw
    phi1 = jnp.dot(rb, wrb1_ref[...], precision=_HIGH) * sw
    phi0_ref[...] = jnp.concatenate([phi0, swb], axis=1)
    phi1_ref[...] = jnp.concatenate([phi1, swb], axis=1)


def _prep(dist_p, sw_p, W_rb_0, W_rb_1):
    B = 2048
    grid = (EP // B,)
    return pl.pallas_call(
        _prep_body,
        grid=grid,
        in_specs=[
            pl.BlockSpec((B, 1), lambda i: (i, 0)),
            pl.BlockSpec((B, 1), lambda i: (i, 0)),
            pl.BlockSpec((RB, DIM_SRC), lambda i: (0, 0)),
            pl.BlockSpec((RB, DIM_SRC), lambda i: (0, 0)),
        ],
        out_specs=[
            pl.BlockSpec((B, UDW), lambda i: (i, 0)),
            pl.BlockSpec((B, UDW), lambda i: (i, 0)),
        ],
        out_shape=[
            jax.ShapeDtypeStruct((EP, UDW), jnp.float32),
            jax.ShapeDtypeStruct((EP, UDW), jnp.float32),
        ],
    )(dist_p, sw_p, W_rb_0, W_rb_1)


def _ud0_body(spec_ref, wud_ref, out_ref):
    oh = (spec_ref[...] == jax.lax.broadcasted_iota(jnp.int32, (N, ZMAX), 1))
    out_ref[...] = jnp.dot(oh.astype(jnp.float32), wud_ref[...], precision=_HIGH)


def _ud0(species2d, W_ud_0):
    return pl.pallas_call(
        _ud0_body,
        out_shape=jax.ShapeDtypeStruct((N, UDW), jnp.float32),
    )(species2d, W_ud_0)


def _edge_body(ud_hbm, src_hbm, dst_hbm, phi_hbm, out_hbm, acc,
               srcb0, srcb1, srcb2, srcb3, dstb0, dstb1, dstb2, dstb3,
               phib0, phib1, rows0, rows1, sbuf0, sbuf1,
               sg0, sg1, sp0, sp1, ss0, ss1,
               ssr0, ssr1, ssr2, ssr3, sds0, sds1, sds2, sds3):
    srcb = [srcb0, srcb1, srcb2, srcb3]
    dstb = [dstb0, dstb1, dstb2, dstb3]
    phib = [phib0, phib1]
    rows = [rows0, rows1]
    sbuf = [sbuf0, sbuf1]
    sem_g = [sg0, sg1]
    sem_p = [sp0, sp1]
    sem_s = [ss0, ss1]
    sem_sr = [ssr0, ssr1, ssr2, ssr3]
    sem_ds = [sds0, sds1, sds2, sds3]

    c = lax.axis_index("c")
    s = lax.axis_index("s")
    w = c * NS + s
    base = w * EPT

    def idx_copy(hbm, e0, buf, sem):
        pltpu.async_copy(hbm.at[pl.ds(e0, CH)], buf, sem)

    def idx_wait(hbm, e0, buf, sem):
        pltpu.make_async_copy(hbm.at[pl.ds(e0, CH)], buf, sem).wait()

    @pl.loop(0, ZR)
    def _(i):
        for jj in range(UDW // LANES):
            sbuf0[i, pl.ds(jj * LANES, LANES)] = jnp.zeros((LANES,), jnp.float32)

    for k in range(NP // NS // ZR):
        pltpu.sync_copy(sbuf0.at[pl.ds(0, ZR)],
                        acc.at[pl.ds(s * (NP // NS) + k * ZR, ZR)])
    plsc.subcore_barrier()

    for j in range(4):
        idx_copy(src_hbm, base + j * CH, srcb[j], sem_sr[j])
    for j in range(2):
        idx_copy(dst_hbm, base + j * CH, dstb[j], sem_ds[j])
        pltpu.async_copy(phi_hbm.at[pl.ds(base + j * CH, CH)], phib[j],
                         sem_p[j])
    for j in range(2):
        idx_wait(src_hbm, base + j * CH, srcb[j], sem_sr[j])
        pltpu.async_copy(ud_hbm.at[srcb[j]], rows[j], sem_g[j])

    @pl.loop(0, CPT // 4)
    def _(u):
        for j in range(4):
            b = j & 1
            q = j
            q2 = (j + 2) & 3
            i = u * 4 + j
            e0 = base + i * CH
            e2 = e0 + 2 * CH
            e4 = e0 + 4 * CH

            idx_wait(phi_hbm, e0, phib[b], sem_p[b])
            pltpu.make_async_copy(ud_hbm.at[srcb[q]],
                                  rows[b], sem_g[b]).wait()

            def wait_scat_prev():
                pltpu.make_async_copy(sbuf[b], acc.at[dstb[q2]],
                                      sem_s[b]).wait()

            if j >= 2:
                wait_scat_prev()
            else:
                pl.when(u >= 1)(wait_scat_prev)

            def issue_dst_next():
                idx_copy(dst_hbm, e2, dstb[q2], sem_ds[q2])

            if j < 2:
                issue_dst_next()
            else:
                pl.when(u < CPT // 4 - 1)(issue_dst_next)

            pl.when(u < CPT // 4 - 1)(
                lambda: idx_copy(src_hbm, e4, srcb[q], sem_sr[q]))

            @pl.loop(0, CH, step=4)
            def _(e):
                for ee in range(4):
                    for jj in range(UDW // LANES):
                        sl = pl.ds(jj * LANES, LANES)
                        sbuf[b][e + ee, sl] = rows[b][e + ee, sl] * phib[b][e + ee, sl]

            idx_wait(dst_hbm, e0, dstb[q], sem_ds[q])
            pltpu.async_copy(sbuf[b], acc.at[dstb[q]], sem_s[b], add=True)

            def prefetch_next():
                idx_wait(src_hbm, e2, srcb[q2], sem_sr[q2])
                pltpu.async_copy(ud_hbm.at[srcb[q2]], rows[b], sem_g[b])
                pltpu.async_copy(phi_hbm.at[pl.ds(e2, CH)], phib[b], sem_p[b])

            if j < 2:
                prefetch_next()
            else:
                pl.when(u < CPT // 4 - 1)(prefetch_next)

    for j in range(2):
        b = (CPT - 2 + j) & 1
        q = (CPT - 2 + j) & 3
        pltpu.make_async_copy(sbuf[b], acc.at[dstb[q]], sem_s[b]).wait()

    plsc.subcore_barrier()
    for k in range(NP // NS // ZR):
        r0 = s * (NP // NS) + k * ZR
        pltpu.sync_copy(acc.at[pl.ds(r0, ZR)], out_hbm.at[c, pl.ds(r0, ZR)])


def _edge_sc(UD, src_p, dst_p, PHI):
    mesh = plsc.VectorSubcoreMesh(core_axis_name="c", subcore_axis_name="s")
    kern = pl.kernel(
        _edge_body,
        out_type=jax.ShapeDtypeStruct((NC, NP, UDW), jnp.float32),
        mesh=mesh,
        compiler_params=pltpu.CompilerParams(use_tc_tiling_on_sc=False),
        scratch_types=[
            pltpu.VMEM_SHARED((NP, UDW), jnp.float32),
            pltpu.VMEM((CH,), jnp.int32),
            pltpu.VMEM((CH,), jnp.int32),
            pltpu.VMEM((CH,), jnp.int32),
            pltpu.VMEM((CH,), jnp.int32),
            pltpu.VMEM((CH,), jnp.int32),
            pltpu.VMEM((CH,), jnp.int32),
            pltpu.VMEM((CH,), jnp.int32),
            pltpu.VMEM((CH,), jnp.int32),
            pltpu.VMEM((CH, UDW), jnp.float32),
            pltpu.VMEM((CH, UDW), jnp.float32),
            pltpu.VMEM((CH, UDW), jnp.float32),
            pltpu.VMEM((CH, UDW), jnp.float32),
            pltpu.VMEM((CH, UDW), jnp.float32),
            pltpu.VMEM((CH, UDW), jnp.float32),
        ] + [pltpu.SemaphoreType.DMA] * 14,
    )
    return kern(UD, src_p, dst_p, PHI)


def _silu_tssr2(x):
    out = x * jax.lax.logistic(x)
    ax = jnp.abs(out)
    return jnp.where(ax <= 1.0,
                     out,
                     jnp.sign(out) * (2.0 * jnp.sqrt(jnp.maximum(ax, 1.0)) - 1.0))


BN = 2000


def _dense0_body(spec_ref, m_ref, w0a_ref, w0b_ref, w0c_ref, b0_ref, wud1_ref,
                 xi1_ref, ud1_ref):
    oh = (spec_ref[...] == jax.lax.broadcasted_iota(jnp.int32, (BN, ZMAX), 1))
    msum = m_ref[0] + m_ref[1]
    pre = (jnp.dot(oh.astype(jnp.float32), w0a_ref[...], precision=_HIGH)
           + jnp.dot(msum[:, :DIM_SRC], w0b_ref[...], precision=_HIGH)
           + jnp.dot(msum[:, DIM_SRC:], w0c_ref[...], precision=_HIGH)
           + b0_ref[...])
    xi1 = _silu_tssr2(pre)
    xi1_ref[...] = xi1
    ud1_ref[...] = jnp.dot(xi1, wud1_ref[...], precision=_HIGH)


def _dense0(species2d, M0, W0a, W0b, W0c, b0, W_ud_1):
    return pl.pallas_call(
        _dense0_body,
        grid=(N // BN,),
        in_specs=[
            pl.BlockSpec((BN, 1), lambda i: (i, 0)),
            pl.BlockSpec((NC, BN, UDW), lambda i: (0, i, 0)),
            pl.BlockSpec((ZMAX, DIM), lambda i: (0, 0)),
            pl.BlockSpec((DIM_SRC, DIM), lambda i: (0, 0)),
            pl.BlockSpec((DIM_DST, DIM), lambda i: (0, 0)),
            pl.BlockSpec((1, DIM), lambda i: (0, 0)),
            pl.BlockSpec((DIM, UDW), lambda i: (0, 0)),
        ],
        out_specs=[
            pl.BlockSpec((BN, DIM), lambda i: (i, 0)),
            pl.BlockSpec((BN, UDW), lambda i: (i, 0)),
        ],
        out_shape=[
            jax.ShapeDtypeStruct((N, DIM), jnp.float32),
            jax.ShapeDtypeStruct((N, UDW), jnp.float32),
        ],
    )(species2d, M0, W0a, W0b, W0c, b0, W_ud_1)


def _dense1_body(xi_ref, m_ref, w1a_ref, w1b_ref, w1c_ref, b1_ref, out_ref):
    xi = xi_ref[...]
    msum = m_ref[0] + m_ref[1]
    pre = (jnp.dot(xi, w1a_ref[...], precision=_HIGH)
           + jnp.dot(msum[:, :DIM_SRC], w1b_ref[...], precision=_HIGH)
           + jnp.dot(msum[:, DIM_SRC:], w1c_ref[...], precision=_HIGH)
           + b1_ref[...])
    out_ref[...] = xi + _silu_tssr2(pre)


def _dense1(xi1, M1, W1a, W1b, W1c, b1):
    return pl.pallas_call(
        _dense1_body,
        grid=(N // BN,),
        in_specs=[
            pl.BlockSpec((BN, DIM), lambda i: (i, 0)),
            pl.BlockSpec((NC, BN, UDW), lambda i: (0, i, 0)),
            pl.BlockSpec((DIM, DIM), lambda i: (0, 0)),
            pl.BlockSpec((DIM_SRC, DIM), lambda i: (0, 0)),
            pl.BlockSpec((DIM_DST, DIM), lambda i: (0, 0)),
            pl.BlockSpec((1, DIM), lambda i: (0, 0)),
        ],
        out_specs=pl.BlockSpec((BN, DIM), lambda i: (i, 0)),
        out_shape=jax.ShapeDtypeStruct((N, DIM), jnp.float32),
    )(xi1, M1, W1a, W1b, W1c, b1)


def kernel(species, edge_src, edge_dst, distances, switch,
           W_ud_0, W_rb_0, W_mix_0, b_mix_0,
           W_ud_1, W_rb_1, W_mix_1, b_mix_1):
    pad = EP - E
    src_p = jnp.pad(edge_src.astype(jnp.int32), (0, pad))
    dst_p = jnp.pad(edge_dst.astype(jnp.int32), (0, pad))
    dist_p = jnp.pad(distances, (0, pad), constant_values=1.0)[:, None]
    sw_p = jnp.pad(switch, (0, pad))[:, None]
    species2d = species.astype(jnp.int32)[:, None]

    PHI0, PHI1 = _prep(dist_p, sw_p, W_rb_0, W_rb_1)
    UD0 = _ud0(species2d, W_ud_0)

    M0 = _edge_sc(UD0, src_p, dst_p, PHI0)
    xi1, UD1 = _dense0(species2d, M0,
                       W_mix_0[:ZMAX], W_mix_0[ZMAX:ZMAX + DIM_SRC],
                       W_mix_0[ZMAX + DIM_SRC:], b_mix_0[None, :], W_ud_1)

    M1 = _edge_sc(UD1, src_p, dst_p, PHI1)
    out = _dense1(xi1, M1,
                  W_mix_1[:DIM], W_mix_1[DIM:DIM + DIM_SRC],
                  W_mix_1[DIM + DIM_SRC:], b_mix_1[None, :])
    return out


# speedup vs baseline: 6.1946x; 2.3746x over previous
import functools

import jax
import jax.numpy as jnp
from jax import lax
from jax.experimental import pallas as pl
from jax.experimental.pallas import tpu as pltpu
from jax.experimental.pallas import tpu_sc as plsc

N = 10000
E = 320000
DIM = 128
DIM_SRC = 64
DIM_DST = 32
UDW = DIM_SRC + DIM_DST
ZMAX = 87
RB = 8
CUTOFF = 5.0

NC, NS, LANES = 2, 16, 16
NW = NC * NS
CH = 128
CPT = 80
EPT = CH * CPT
EP = NW * EPT
EPS = EP * RB
RBROWS = EPS // 128
NP = 10240
ZR = 80

_HIGH = jax.lax.Precision.HIGHEST


def _prep_body(xr_ref, swr_ref, rb_ref):
    xr = xr_ref[...]
    swr = swr_ref[...]
    pat = ((jax.lax.broadcasted_iota(jnp.int32, (1, 128), 1) % RB) + 1
           ).astype(jnp.float32)
    arg = xr * (pat * (jnp.pi / CUTOFF))
    rb_ref[...] = (jnp.sqrt(2.0 / CUTOFF) * jnp.sin(arg)
                   / jnp.maximum(xr, 1e-3)) * swr


def _prep(xrep, swrep):
    RBLK = RBROWS // 16
    return pl.pallas_call(
        _prep_body,
        grid=(16,),
        in_specs=[
            pl.BlockSpec((RBLK, 128), lambda i: (i, 0)),
            pl.BlockSpec((RBLK, 128), lambda i: (i, 0)),
        ],
        out_specs=pl.BlockSpec((RBLK, 128), lambda i: (i, 0)),
        out_shape=jax.ShapeDtypeStruct((RBROWS, 128), jnp.float32),
    )(xrep, swrep)


def _ud0_body(spec_ref, wud_ref, out_ref):
    oh = (spec_ref[...] == jax.lax.broadcasted_iota(jnp.int32, (N, ZMAX), 1))
    out_ref[...] = jnp.dot(oh.astype(jnp.float32), wud_ref[...], precision=_HIGH)


def _ud0(species2d, W_ud_0):
    return pl.pallas_call(
        _ud0_body,
        out_shape=jax.ShapeDtypeStruct((N, UDW), jnp.float32),
    )(species2d, W_ud_0)


def _edge_body(ud_hbm, src_hbm, dst_hbm, rb_hbm, sw_hbm, wrb_hbm, out_hbm, acc,
               srcb0, srcb1, srcb2, srcb3, dstb0, dstb1, dstb2, dstb3,
               rbb0, rbb1, swb0, swb1, rows0, rows1, sbuf0, sbuf1, wtab,
               sg0, sg1, sp0, sp1, sw0, sw1, ss0, ss1,
               ssr0, ssr1, ssr2, ssr3, sds0, sds1, sds2, sds3, swt):
    srcb = [srcb0, srcb1, srcb2, srcb3]
    dstb = [dstb0, dstb1, dstb2, dstb3]
    rbb = [rbb0, rbb1]
    swb = [swb0, swb1]
    rows = [rows0, rows1]
    sbuf = [sbuf0, sbuf1]
    sem_g = [sg0, sg1]
    sem_p = [sp0, sp1]
    sem_w = [sw0, sw1]
    sem_s = [ss0, ss1]
    sem_sr = [ssr0, ssr1, ssr2, ssr3]
    sem_ds = [sds0, sds1, sds2, sds3]

    c = lax.axis_index("c")
    s = lax.axis_index("s")
    w = c * NS + s
    base = w * EPT

    def idx_copy(hbm, e0, buf, sem):
        pltpu.async_copy(hbm.at[pl.ds(e0, CH)], buf, sem)

    def idx_wait(hbm, e0, buf, sem):
        pltpu.make_async_copy(hbm.at[pl.ds(e0, CH)], buf, sem).wait()

    def rb_copy(e0, buf, sem):
        pltpu.async_copy(rb_hbm.at[pl.ds(e0 * RB, CH * RB)], buf, sem)

    def rb_wait(e0, buf, sem):
        pltpu.make_async_copy(rb_hbm.at[pl.ds(e0 * RB, CH * RB)], buf,
                              sem).wait()

    pltpu.async_copy(wrb_hbm, wtab, swt)
    pltpu.make_async_copy(wrb_hbm, wtab, swt).wait()
    wv = [[wtab[b, pl.ds(kk * LANES, LANES)] for kk in range(DIM_SRC // LANES)]
          for b in range(RB)]

    @pl.loop(0, ZR)
    def _(i):
        for jj in range(UDW // LANES):
            sbuf0[i, pl.ds(jj * LANES, LANES)] = jnp.zeros((LANES,), jnp.float32)

    for k in range(NP // NS // ZR):
        pltpu.sync_copy(sbuf0.at[pl.ds(0, ZR)],
                        acc.at[pl.ds(s * (NP // NS) + k * ZR, ZR)])
    plsc.subcore_barrier()

    for j in range(4):
        idx_copy(src_hbm, base + j * CH, srcb[j], sem_sr[j])
    for j in range(2):
        idx_copy(dst_hbm, base + j * CH, dstb[j], sem_ds[j])
        rb_copy(base + j * CH, rbb[j], sem_p[j])
        idx_copy(sw_hbm, base + j * CH, swb[j], sem_w[j])
    for j in range(2):
        idx_wait(src_hbm, base + j * CH, srcb[j], sem_sr[j])
        pltpu.async_copy(ud_hbm.at[srcb[j]], rows[j], sem_g[j])

    @pl.loop(0, CPT // 4)
    def _(u):
        for j in range(4):
            b = j & 1
            q = j
            q2 = (j + 2) & 3
            i = u * 4 + j
            e0 = base + i * CH
            e2 = e0 + 2 * CH
            e4 = e0 + 4 * CH

            rb_wait(e0, rbb[b], sem_p[b])
            idx_wait(sw_hbm, e0, swb[b], sem_w[b])
            pltpu.make_async_copy(ud_hbm.at[srcb[q]],
                                  rows[b], sem_g[b]).wait()

            def wait_scat_prev():
                pltpu.make_async_copy(sbuf[b], acc.at[dstb[q2]],
                                      sem_s[b]).wait()

            if j >= 2:
                wait_scat_prev()
            else:
                pl.when(u >= 1)(wait_scat_prev)

            def issue_dst_next():
                idx_copy(dst_hbm, e2, dstb[q2], sem_ds[q2])

            if j < 2:
                issue_dst_next()
            else:
                pl.when(u < CPT // 4 - 1)(issue_dst_next)

            pl.when(u < CPT // 4 - 1)(
                lambda: idx_copy(src_hbm, e4, srcb[q], sem_sr[q]))

            @pl.loop(0, CH, step=LANES)
            def _(e):
                swvec = swb[b][pl.ds(e, LANES)]
                for ee in range(LANES):
                    eg = e + ee
                    rv = rbb[b][pl.ds(e * RB + ee * RB, LANES)]
                    swv = swvec[ee]
                    for kk in range(DIM_SRC // LANES):
                        wk = rv[0] * wv[0][kk]
                        for bb in range(1, RB):
                            wk = wk + rv[bb] * wv[bb][kk]
                        sl = pl.ds(kk * LANES, LANES)
                        sbuf[b][eg, sl] = rows[b][eg, sl] * wk
                    for kk in range(UDW // LANES - DIM_SRC // LANES):
                        sl = pl.ds(DIM_SRC + kk * LANES, LANES)
                        sbuf[b][eg, sl] = rows[b][eg, sl] * swv

            idx_wait(dst_hbm, e0, dstb[q], sem_ds[q])
            pltpu.async_copy(sbuf[b], acc.at[dstb[q]], sem_s[b], add=True)

            def prefetch_next():
                idx_wait(src_hbm, e2, srcb[q2], sem_sr[q2])
                pltpu.async_copy(ud_hbm.at[srcb[q2]], rows[b], sem_g[b])
                rb_copy(e2, rbb[b], sem_p[b])
                idx_copy(sw_hbm, e2, swb[b], sem_w[b])

            if j < 2:
                prefetch_next()
            else:
                pl.when(u < CPT // 4 - 1)(prefetch_next)

    for j in range(2):
        b = (CPT - 2 + j) & 1
        q = (CPT - 2 + j) & 3
        pltpu.make_async_copy(sbuf[b], acc.at[dstb[q]], sem_s[b]).wait()

    plsc.subcore_barrier()
    for k in range(NP // NS // ZR):
        r0 = s * (NP // NS) + k * ZR
        pltpu.sync_copy(acc.at[pl.ds(r0, ZR)], out_hbm.at[c, pl.ds(r0, ZR)])


def _edge_sc(UD, src_p, dst_p, RBF, sw1, W_rb):
    mesh = plsc.VectorSubcoreMesh(core_axis_name="c", subcore_axis_name="s")
    kern = pl.kernel(
        _edge_body,
        out_type=jax.ShapeDtypeStruct((NC, NP, UDW), jnp.float32),
        mesh=mesh,
        compiler_params=pltpu.CompilerParams(use_tc_tiling_on_sc=False),
        scratch_types=[
            pltpu.VMEM_SHARED((NP, UDW), jnp.float32),
            pltpu.VMEM((CH,), jnp.int32),
            pltpu.VMEM((CH,), jnp.int32),
            pltpu.VMEM((CH,), jnp.int32),
            pltpu.VMEM((CH,), jnp.int32),
            pltpu.VMEM((CH,), jnp.int32),
            pltpu.VMEM((CH,), jnp.int32),
            pltpu.VMEM((CH,), jnp.int32),
            pltpu.VMEM((CH,), jnp.int32),
            pltpu.VMEM((CH * RB,), jnp.float32),
            pltpu.VMEM((CH * RB,), jnp.float32),
            pltpu.VMEM((CH,), jnp.float32),
            pltpu.VMEM((CH,), jnp.float32),
            pltpu.VMEM((CH, UDW), jnp.float32),
            pltpu.VMEM((CH, UDW), jnp.float32),
            pltpu.VMEM((CH, UDW), jnp.float32),
            pltpu.VMEM((CH, UDW), jnp.float32),
            pltpu.VMEM((RB, DIM_SRC), jnp.float32),
        ] + [pltpu.SemaphoreType.DMA] * 17,
    )
    return kern(UD, src_p, dst_p, RBF, sw1, W_rb)


def _silu_tssr2(x):
    out = x * jax.lax.logistic(x)
    ax = jnp.abs(out)
    return jnp.where(ax <= 1.0,
                     out,
                     jnp.sign(out) * (2.0 * jnp.sqrt(jnp.maximum(ax, 1.0)) - 1.0))


BN = 2000


def _dense0_body(spec_ref, m_ref, w0a_ref, w0b_ref, w0c_ref, b0_ref, wud1_ref,
                 xi1_ref, ud1_ref):
    oh = (spec_ref[...] == jax.lax.broadcasted_iota(jnp.int32, (BN, ZMAX), 1))
    msum = m_ref[0] + m_ref[1]
    pre = (jnp.dot(oh.astype(jnp.float32), w0a_ref[...], precision=_HIGH)
           + jnp.dot(msum[:, :DIM_SRC], w0b_ref[...], precision=_HIGH)
           + jnp.dot(msum[:, DIM_SRC:], w0c_ref[...], precision=_HIGH)
           + b0_ref[...])
    xi1 = _silu_tssr2(pre)
    xi1_ref[...] = xi1
    ud1_ref[...] = jnp.dot(xi1, wud1_ref[...], precision=_HIGH)


def _dense0(species2d, M0, W0a, W0b, W0c, b0, W_ud_1):
    return pl.pallas_call(
        _dense0_body,
        grid=(N // BN,),
        in_specs=[
            pl.BlockSpec((BN, 1), lambda i: (i, 0)),
            pl.BlockSpec((NC, BN, UDW), lambda i: (0, i, 0)),
            pl.BlockSpec((ZMAX, DIM), lambda i: (0, 0)),
            pl.BlockSpec((DIM_SRC, DIM), lambda i: (0, 0)),
            pl.BlockSpec((DIM_DST, DIM), lambda i: (0, 0)),
            pl.BlockSpec((1, DIM), lambda i: (0, 0)),
            pl.BlockSpec((DIM, UDW), lambda i: (0, 0)),
        ],
        out_specs=[
            pl.BlockSpec((BN, DIM), lambda i: (i, 0)),
            pl.BlockSpec((BN, UDW), lambda i: (i, 0)),
        ],
        out_shape=[
            jax.ShapeDtypeStruct((N, DIM), jnp.float32),
            jax.ShapeDtypeStruct((N, UDW), jnp.float32),
        ],
    )(species2d, M0, W0a, W0b, W0c, b0, W_ud_1)


def _dense1_body(xi_ref, m_ref, w1a_ref, w1b_ref, w1c_ref, b1_ref, out_ref):
    xi = xi_ref[...]
    msum = m_ref[0] + m_ref[1]
    pre = (jnp.dot(xi, w1a_ref[...], precision=_HIGH)
           + jnp.dot(msum[:, :DIM_SRC], w1b_ref[...], precision=_HIGH)
           + jnp.dot(msum[:, DIM_SRC:], w1c_ref[...], precision=_HIGH)
           + b1_ref[...])
    out_ref[...] = xi + _silu_tssr2(pre)


def _dense1(xi1, M1, W1a, W1b, W1c, b1):
    return pl.pallas_call(
        _dense1_body,
        grid=(N // BN,),
        in_specs=[
            pl.BlockSpec((BN, DIM), lambda i: (i, 0)),
            pl.BlockSpec((NC, BN, UDW), lambda i: (0, i, 0)),
            pl.BlockSpec((DIM, DIM), lambda i: (0, 0)),
            pl.BlockSpec((DIM_SRC, DIM), lambda i: (0, 0)),
            pl.BlockSpec((DIM_DST, DIM), lambda i: (0, 0)),
            pl.BlockSpec((1, DIM), lambda i: (0, 0)),
        ],
        out_specs=pl.BlockSpec((BN, DIM), lambda i: (i, 0)),
        out_shape=jax.ShapeDtypeStruct((N, DIM), jnp.float32),
    )(xi1, M1, W1a, W1b, W1c, b1)


def kernel(species, edge_src, edge_dst, distances, switch,
           W_ud_0, W_rb_0, W_mix_0, b_mix_0,
           W_ud_1, W_rb_1, W_mix_1, b_mix_1):
    pad = EP - E
    src_p = jnp.pad(edge_src.astype(jnp.int32), (0, pad))
    dst_p = jnp.pad(edge_dst.astype(jnp.int32), (0, pad))
    dist_p = jnp.pad(distances, (0, pad), constant_values=1.0)
    sw_p = jnp.pad(switch, (0, pad))
    xrep = jnp.broadcast_to(dist_p[:, None], (EP, RB)).reshape(RBROWS, 128)
    swrep = jnp.broadcast_to(sw_p[:, None], (EP, RB)).reshape(RBROWS, 128)
    species2d = species.astype(jnp.int32)[:, None]

    RBF = _prep(xrep, swrep).reshape(EPS)
    UD0 = _ud0(species2d, W_ud_0)

    M0 = _edge_sc(UD0, src_p, dst_p, RBF, sw_p, W_rb_0)
    xi1, UD1 = _dense0(species2d, M0,
                       W_mix_0[:ZMAX], W_mix_0[ZMAX:ZMAX + DIM_SRC],
                       W_mix_0[ZMAX + DIM_SRC:], b_mix_0[None, :], W_ud_1)

    M1 = _edge_sc(UD1, src_p, dst_p, RBF, sw_p, W_rb_1)
    out = _dense1(xi1, M1,
                  W_mix_1[:DIM], W_mix_1[DIM:DIM + DIM_SRC],
                  W_mix_1[DIM + DIM_SRC:], b_mix_1[None, :])
    return out
